# baseline (speedup 1.0000x reference)
import jax
import jax.numpy as jnp
from jax.experimental import pallas as pl

N = 10000
D = 128
H1 = 3
C1 = 64
C2 = 128


def _norm_tail_kernel(out_ref, res_ref, gamma_ref, beta_ref, o_ref):
    out = out_ref[...]
    mean = jnp.mean(out, axis=0, keepdims=True)
    var = jnp.mean((out - mean) ** 2, axis=0, keepdims=True)
    y = (out - mean) / jnp.sqrt(var + 1e-5) * gamma_ref[...] + beta_ref[...]
    o_ref[...] = jnp.maximum(y, 0.0) + res_ref[...]


def _gat(x, W, a_src, a_dst, b, src, dst, heads, ch):
    n = x.shape[0]
    h = (x @ W).reshape(n, heads, ch)
    alpha_src = (h * a_src[None]).sum(-1)
    alpha_dst = (h * a_dst[None]).sum(-1)
    e = jax.nn.leaky_relu(alpha_src[src] + alpha_dst[dst], 0.2)
    m = jax.ops.segment_max(e, dst, num_segments=n)
    m = jnp.where(jnp.isfinite(m), m, 0.0)
    ex = jnp.exp(e - m[dst])
    s = jax.ops.segment_sum(ex, dst, num_segments=n)
    alpha = ex / (s[dst] + 1e-16)
    out = jax.ops.segment_sum(h[src] * alpha[..., None], dst, num_segments=n)
    return out.reshape(n, heads * ch) + b


def kernel(x, edge_index, W1, a1_src, a1_dst, b1, W2, a2_src, a2_dst, b2, Wr, br, gamma, beta):
    n = x.shape[0]
    loop = jnp.arange(n, dtype=edge_index.dtype)
    src = jnp.concatenate([edge_index[0], loop])
    dst = jnp.concatenate([edge_index[1], loop])
    residual = x @ Wr + br
    out = _gat(x, W1, a1_src, a1_dst, b1, src, dst, H1, C1)
    out = _gat(out, W2, a2_src, a2_dst, b2, src, dst, 1, C2)
    return pl.pallas_call(
        _norm_tail_kernel,
        out_shape=jax.ShapeDtypeStruct((n, C2), jnp.float32),
    )(out, residual, gamma.reshape(1, C2), beta.reshape(1, C2))


# speedup vs baseline: 13.0945x; 13.0945x over previous
import functools

import jax
import jax.numpy as jnp
from jax import lax
from jax.experimental import pallas as pl
from jax.experimental.pallas import tpu as pltpu
from jax.experimental.pallas import tpu_sc as plsc

N = 10000
E = 320000
D = 128
H1 = 3
C1 = 64
C2 = 128
W1COLS = H1 * C1
DH1 = W1COLS // 2
WR1 = DH1 + 16
DH2 = C2 // 2
WR2 = DH2 + 16
NC = 2
NS = 16
EPT = E // NS
K = 80
NCHUNK = EPT // K
RPT = N // NS


NB = 1000
G = N // NB


def _tc_pre(x_ref, w1_ref, a1s_ref, a1d_ref, wr_ref, br_ref,
            h1p_ref, as1_ref, ad1_ref, acc0_ref, res_ref):
    x = x_ref[...]
    h1 = jnp.dot(x, w1_ref[...], preferred_element_type=jnp.float32)
    h3 = h1.reshape(NB, H1, C1)
    as1 = (h3 * a1s_ref[...][None]).sum(-1)
    ad1 = (h3 * a1d_ref[...][None]).sum(-1)
    e = as1 + ad1
    wself = jnp.exp(jnp.where(e >= 0.0, e, 0.2 * e))
    z16 = jnp.zeros((NB, 16), jnp.float32)
    zp = jnp.zeros((NB, WR1 - DH1 - H1), jnp.float32)
    h1p_ref[...] = jnp.stack(
        [jnp.concatenate([h1[:, :DH1], z16], 1),
         jnp.concatenate([h1[:, DH1:], z16], 1)])
    as1_ref[...] = as1.reshape(1, 1, H1 * NB)
    ad1_ref[...] = ad1.reshape(1, 1, H1 * NB)
    hw = (h3 * wself[..., None]).reshape(NB, W1COLS)
    acc0_ref[...] = jnp.stack(
        [jnp.concatenate([hw[:, :DH1], wself, zp], 1),
         jnp.concatenate([hw[:, DH1:], wself, zp], 1)])
    res_ref[...] = jnp.dot(x, wr_ref[...],
                           preferred_element_type=jnp.float32) + br_ref[...]


def _tc_mid(acc_ref, b1_ref, w2_ref, a2s_ref, a2d_ref,
            h2p_ref, as2_ref, ad2_ref, acc20_ref):
    num = jnp.concatenate([acc_ref[0, :, :DH1], acc_ref[1, :, :DH1]], 1)
    s = acc_ref[0, :, DH1:DH1 + H1]
    out1 = (num.reshape(NB, H1, C1) / s[..., None]).reshape(NB, W1COLS)
    out1 = out1 + b1_ref[...]
    h2 = jnp.dot(out1, w2_ref[...], preferred_element_type=jnp.float32)
    as2 = (h2 * a2s_ref[...]).sum(-1, keepdims=True)
    ad2 = (h2 * a2d_ref[...]).sum(-1, keepdims=True)
    e = as2 + ad2
    wself = jnp.exp(jnp.where(e >= 0.0, e, 0.2 * e))
    z16 = jnp.zeros((NB, 16), jnp.float32)
    zp = jnp.zeros((NB, WR2 - DH2 - 1), jnp.float32)
    h2p_ref[...] = jnp.stack(
        [jnp.concatenate([h2[:, :DH2], z16], 1),
         jnp.concatenate([h2[:, DH2:], z16], 1)])
    as2_ref[...] = as2.reshape(1, 1, NB)
    ad2_ref[...] = ad2.reshape(1, 1, NB)
    hw = h2 * wself
    acc20_ref[...] = jnp.stack(
        [jnp.concatenate([hw[:, :DH2], wself, zp], 1),
         jnp.concatenate([hw[:, DH2:], wself, zp], 1)])


def _tc_fin_a(acc_ref, b2_ref, out_ref, st_ref):
    num = jnp.concatenate([acc_ref[0, :, :DH2], acc_ref[1, :, :DH2]], 1)
    s = acc_ref[0, :, DH2:DH2 + 1]
    out = num / s + b2_ref[...]
    out_ref[...] = out

    @pl.when(pl.program_id(0) == 0)
    def _():
        st_ref[...] = jnp.zeros_like(st_ref)

    st_ref[...] = st_ref[...] + jnp.stack(
        [out.sum(0), (out * out).sum(0)])


def _tc_fin_b(out_ref, st_ref, g_ref, bt_ref, res_ref, o_ref):
    out = out_ref[...]
    mean = st_ref[0:1, :] / N
    var = st_ref[1:2, :] / N - mean * mean
    y = (out - mean) / jnp.sqrt(var + 1e-5) * g_ref[...] + bt_ref[...]
    o_ref[...] = jnp.maximum(y, 0.0) + res_ref[...]


def _make_sc_edge(heads, dhalf, wr):
    mesh = plsc.VectorSubcoreMesh(core_axis_name="c", subcore_axis_name="s")

    @functools.partial(
        pl.kernel,
        out_type=jax.ShapeDtypeStruct((NC * N, wr), jnp.float32),
        mesh=mesh,
        compiler_params=pltpu.CompilerParams(
            use_tc_tiling_on_sc=False, needs_layout_passes=False),
        scratch_types=[
            pltpu.VMEM((K,), jnp.int32),
            pltpu.VMEM((K,), jnp.int32),
            [pltpu.VMEM((K,), jnp.int32) for _ in range(heads)],
            [pltpu.VMEM((K,), jnp.int32) for _ in range(heads)],
            [pltpu.VMEM((K,), jnp.float32) for _ in range(heads)],
            [pltpu.VMEM((K,), jnp.float32) for _ in range(heads)],
            pltpu.VMEM((K, wr), jnp.float32),
            pltpu.VMEM_SHARED((N, wr), jnp.float32),
            pltpu.SemaphoreType.DMA,
        ],
    )
    def sc_edge(h_hbm, as_hbm, ad_hbm, edge_hbm, gsrc_hbm, sidx_hbm,
                didx_hbm, init_hbm, out_hbm,
                dst_v, gidx_v, aidx, didx, asv, adv,
                rows_v, acc_sh, sem):
        cid = lax.axis_index("c")
        sid = lax.axis_index("s")
        r0 = sid * RPT

        pltpu.sync_copy(init_hbm.at[pl.ds(cid * N + r0, RPT)],
                        acc_sh.at[pl.ds(r0, RPT)])
        plsc.subcore_barrier()

        @pl.loop(0, NCHUNK)
        def chunk(c):
            off = pl.multiple_of(sid * EPT + c * K, 8)
            pltpu.sync_copy(edge_hbm.at[pl.ds(pl.multiple_of(E + off, 8),
                                              K)], dst_v)
            pltpu.sync_copy(gsrc_hbm.at[pl.ds(pl.multiple_of(cid * E + off,
                                                             8), K)], gidx_v)
            for hh in range(heads):
                offh = pl.multiple_of(hh * E + off, 8)
                pltpu.sync_copy(sidx_hbm.at[pl.ds(offh, K)], aidx[hh])
                pltpu.sync_copy(didx_hbm.at[pl.ds(offh, K)], didx[hh])
            rows_cp = pltpu.async_copy(h_hbm.at[gidx_v], rows_v, sem)
            cps = []
            for hh in range(heads):
                cps.append(pltpu.async_copy(as_hbm.at[aidx[hh]],
                                            asv[hh], sem))
                cps.append(pltpu.async_copy(ad_hbm.at[didx[hh]],
                                            adv[hh], sem))
            for cp in cps:
                cp.wait()
            rows_cp.wait()

            def group(j, carry2):
                rvec = j * 16 + lax.iota(jnp.int32, 16)
                wvs = []
                for hh in range(heads):
                    e = (asv[hh][pl.ds(j * 16, 16)]
                         + adv[hh][pl.ds(j * 16, 16)])
                    wvs.append(jnp.exp(jnp.where(e >= 0.0, e, 0.2 * e)))
                if heads == 3:
                    is0 = cid == 0
                    sel = [jnp.where(is0, wvs[0], wvs[1]),
                           jnp.where(is0, wvs[0], wvs[2]),
                           jnp.where(is0, wvs[1], wvs[2])]
                else:
                    sel = [wvs[0]]

                for b in range(dhalf):
                    w = sel[(b * heads) // dhalf] if heads == 3 else sel[0]
                    cvec = jnp.full((16,), b, jnp.int32)
                    r = plsc.load_gather(rows_v, [rvec, cvec])
                    plsc.store_scatter(rows_v, [rvec, cvec], r * w)
                for hh in range(heads):
                    plsc.store_scatter(
                        rows_v,
                        [rvec, jnp.full((16,), dhalf + hh, jnp.int32)],
                        wvs[hh])
                return carry2

            lax.fori_loop(0, K // 16, group, 0)
            pltpu.sync_copy(rows_v, acc_sh.at[dst_v], add=True)
        plsc.subcore_barrier()
        pltpu.sync_copy(acc_sh.at[pl.ds(r0, RPT)],
                        out_hbm.at[pl.ds(cid * N + r0, RPT)])

    return sc_edge


_sc_edge1 = _make_sc_edge(H1, DH1, WR1)
_sc_edge2 = _make_sc_edge(1, DH2, WR2)


def _full(shape):
    return pl.BlockSpec(shape, lambda i: tuple(0 for _ in shape))


def kernel(x, edge_index, W1, a1_src, a1_dst, b1, W2, a2_src, a2_dst, b2,
           Wr, br, gamma, beta):
    f32 = jnp.float32
    h1p, as1, ad1, acc0, resid = pl.pallas_call(
        _tc_pre,
        grid=(G,),
        in_specs=[
            pl.BlockSpec((NB, D), lambda i: (i, 0)),
            _full((D, W1COLS)),
            _full((H1, C1)),
            _full((H1, C1)),
            _full((D, C2)),
            _full((1, C2)),
        ],
        out_specs=[
            pl.BlockSpec((NC, NB, WR1), lambda i: (0, i, 0)),
            pl.BlockSpec((1, 1, H1 * NB), lambda i: (i, 0, 0)),
            pl.BlockSpec((1, 1, H1 * NB), lambda i: (i, 0, 0)),
            pl.BlockSpec((NC, NB, WR1), lambda i: (0, i, 0)),
            pl.BlockSpec((NB, C2), lambda i: (i, 0)),
        ],
        out_shape=[
            jax.ShapeDtypeStruct((NC, N, WR1), f32),
            jax.ShapeDtypeStruct((G, 1, H1 * NB), f32),
            jax.ShapeDtypeStruct((G, 1, H1 * NB), f32),
            jax.ShapeDtypeStruct((NC, N, WR1), f32),
            jax.ShapeDtypeStruct((N, C2), f32),
        ],
    )(x, W1, a1_src, a1_dst, Wr, br.reshape(1, C2))

    eflat = edge_index.reshape(2 * E)
    src_, dst_ = edge_index[0], edge_index[1]
    gsrc = jnp.concatenate([src_, src_ + N])
    sidx1 = jnp.concatenate([src_ * H1, src_ * H1 + 1, src_ * H1 + 2])
    didx1 = jnp.concatenate([dst_ * H1, dst_ * H1 + 1, dst_ * H1 + 2])
    acc1 = _sc_edge1(h1p.reshape(NC * N, WR1), as1.reshape(H1 * N),
                     ad1.reshape(H1 * N), eflat, gsrc, sidx1, didx1,
                     acc0.reshape(NC * N, WR1)).reshape(NC, N, WR1)

    h2p, as2, ad2, acc20 = pl.pallas_call(
        _tc_mid,
        grid=(G,),
        in_specs=[
            pl.BlockSpec((NC, NB, WR1), lambda i: (0, i, 0)),
            _full((1, W1COLS)),
            _full((W1COLS, C2)),
            _full((1, C2)),
            _full((1, C2)),
        ],
        out_specs=[
            pl.BlockSpec((NC, NB, WR2), lambda i: (0, i, 0)),
            pl.BlockSpec((1, 1, NB), lambda i: (i, 0, 0)),
            pl.BlockSpec((1, 1, NB), lambda i: (i, 0, 0)),
            pl.BlockSpec((NC, NB, WR2), lambda i: (0, i, 0)),
        ],
        out_shape=[
            jax.ShapeDtypeStruct((NC, N, WR2), f32),
            jax.ShapeDtypeStruct((G, 1, NB), f32),
            jax.ShapeDtypeStruct((G, 1, NB), f32),
            jax.ShapeDtypeStruct((NC, N, WR2), f32),
        ],
    )(acc1, b1.reshape(1, W1COLS), W2, a2_src, a2_dst)

    acc2 = _sc_edge2(h2p.reshape(NC * N, WR2), as2.reshape(N),
                     ad2.reshape(N), eflat, gsrc, src_, dst_,
                     acc20.reshape(NC * N, WR2)).reshape(NC, N, WR2)

    out2, stats = pl.pallas_call(
        _tc_fin_a,
        grid=(G,),
        in_specs=[
            pl.BlockSpec((NC, NB, WR2), lambda i: (0, i, 0)),
            _full((1, C2)),
        ],
        out_specs=[
            pl.BlockSpec((NB, C2), lambda i: (i, 0)),
            _full((2, C2)),
        ],
        out_shape=[
            jax.ShapeDtypeStruct((N, C2), f32),
            jax.ShapeDtypeStruct((2, C2), f32),
        ],
    )(acc2, b2.reshape(1, C2))

    return pl.pallas_call(
        _tc_fin_b,
        grid=(G,),
        in_specs=[
            pl.BlockSpec((NB, C2), lambda i: (i, 0)),
            _full((2, C2)),
            _full((1, C2)),
            _full((1, C2)),
            pl.BlockSpec((NB, C2), lambda i: (i, 0)),
        ],
        out_specs=pl.BlockSpec((NB, C2), lambda i: (i, 0)),
        out_shape=jax.ShapeDtypeStruct((N, C2), f32),
    )(out2, stats, gamma.reshape(1, C2), beta.reshape(1, C2), resid)
